# baseline (speedup 1.0000x reference)
import functools

import jax
import jax.numpy as jnp
from jax import lax
from jax.experimental import pallas as pl
from jax.experimental.pallas import tpu as pltpu
from jax.experimental.pallas import tpu_sc as plsc

DIM = 32
BATCH = 16384
NROW = 1000000

NC = 2
NS = 16
NW = NC * NS
BPW = BATCH // NW

CHUNK = 512
CPW = 61
FULL = 999936
TAILLO = FULL
PAD0 = BATCH
NSTG = BATCH + NW * 16


def _mesh():
    return plsc.VectorSubcoreMesh(
        core_axis_name="c", subcore_axis_name="s",
        num_cores=NC, num_subcores=NS)


@functools.cache
def _build_sweep():
    return pl.kernel(
        _sweep_body,
        mesh=_mesh(),
        compiler_params=pltpu.CompilerParams(
            needs_layout_passes=False, use_tc_tiling_on_sc=True),
        out_type=(jax.ShapeDtypeStruct((NSTG, 128), jnp.float32),
                  jax.ShapeDtypeStruct((NSTG, 128), jnp.float32)),
        scratch_types=[
            pltpu.VMEM((BATCH,), jnp.int32),
            pltpu.VMEM((BATCH + 16,), jnp.int32),
            pltpu.VMEM((BATCH + 16,), jnp.int32),
            pltpu.VMEM((BATCH + 16,), jnp.int32),
            pltpu.VMEM((BATCH + 16,), jnp.int32),
            pltpu.VMEM((4, 8, CHUNK), jnp.float32),
            pltpu.VMEM((4, 8, CHUNK), jnp.float32),
            pltpu.VMEM((4, 8, 128), jnp.float32),
            pltpu.VMEM((4, 8, 128), jnp.float32),
            pltpu.VMEM((16,), jnp.int32),
            pltpu.VMEM((512,), jnp.float32),
            pltpu.VMEM((16, 128), jnp.float32),
            pltpu.SemaphoreType.DMA,
            pltpu.SemaphoreType.DMA,
            pltpu.SemaphoreType.DMA,
        ],
    )


def _sweep_body(uidx_hbm, iidx_hbm, ut3_hbm, it3_hbm, utail_hbm, itail_hbm,
                uemb_hbm, iemb_hbm,
                idx_v, hitl_v, hitp_v, chl_v, chp_v, chunk_a, chunk_b,
                utail_v, itail_v, posb_v, trans_v, rows_v, sem, sema, semb):
    wid = lax.axis_index("s") * NC + lax.axis_index("c")
    lane16 = lax.iota(jnp.int32, 16)
    pad_pos = PAD0 + wid * 16 + lane16
    lo = wid * (CPW * CHUNK)
    is_last = wid == NW - 1
    hi = jnp.where(is_last, NROW, lo + CPW * CHUNK)

    pltpu.sync_copy(utail_hbm, utail_v)
    pltpu.sync_copy(itail_hbm, itail_v)

    def one_table(idx_hbm, tab_hbm, tail_v, emb_hbm):
        pltpu.sync_copy(idx_hbm, idx_v)

        def filt(t, cnt):
            vec = idx_v[pl.ds(t * 16, 16)]
            m = (vec >= lo) & (vec < hi)
            plsc.store_compressed(hitl_v.at[pl.ds(cnt, 16)], vec, mask=m)
            plsc.store_compressed(
                hitp_v.at[pl.ds(cnt, 16)], t * 16 + lane16, mask=m)
            return cnt + plsc.all_reduce_population_count(m)[0]

        cnt = lax.fori_loop(0, BATCH // 16, filt, jnp.int32(0))
        hitl_v[pl.ds(cnt, 16)] = jnp.broadcast_to(jnp.int32(lo), (16,))
        hitp_v[pl.ds(cnt, 16)] = pad_pos

        def do_extract(src_v, c0, width, cp):
            ngrp = (cnt + 15) // 16

            def refil(g, ccnt):
                v = hitl_v[pl.ds(g * 16, 16)]
                p = hitp_v[pl.ds(g * 16, 16)]
                m = (v >= c0) & (v < c0 + width)
                plsc.store_compressed(
                    chl_v.at[pl.ds(ccnt, 16)], v - c0, mask=m)
                plsc.store_compressed(chp_v.at[pl.ds(ccnt, 16)], p, mask=m)
                return ccnt + plsc.all_reduce_population_count(m)[0]

            ccnt = lax.fori_loop(0, ngrp, refil, jnp.int32(0))
            chl_v[pl.ds(ccnt, 16)] = jnp.zeros((16,), jnp.int32)
            chp_v[pl.ds(ccnt, 16)] = pad_pos
            if cp is not None:
                cp.wait()

            d16 = lane16 * 16

            def ext(h, carry):
                lvec = chl_v[pl.ds(h * 16, 16)]
                posb_v[pl.ds(0, 16)] = chp_v[pl.ds(h * 16, 16)]
                for d in range(DIM):
                    i, s = divmod(d, 8)
                    ivec = jnp.full((16,), i, jnp.int32)
                    svec = jnp.full((16,), s, jnp.int32)
                    vals = plsc.load_gather(src_v, [ivec, svec, lvec])
                    trans_v[pl.ds(d * 16, 16)] = vals
                for k in range(16):
                    rows_v[k, pl.ds(0, 16)] = plsc.load_gather(
                        trans_v, [d16 + k])
                    rows_v[k, pl.ds(16, 16)] = plsc.load_gather(
                        trans_v, [d16 + (256 + k)])
                pltpu.async_copy(rows_v, emb_hbm.at[posb_v], sem).wait()
                return carry

            lax.fori_loop(0, jnp.minimum(ccnt, 0) // 16, ext, 0)

        def start(c0, buf, s):
            c0a = pl.multiple_of(c0, 128)
            cps = [pltpu.async_copy(
                tab_hbm.at[pl.ds(i, 1), :, pl.ds(c0a, CHUNK)],
                buf.at[pl.ds(i, 1)], s) for i in range(4)]
            return cps[-1]

        start(lo, chunk_a, sema)

        def pair_iter(g, carry):
            c0 = lo + (2 * g) * CHUNK
            cpb = start(c0 + CHUNK, chunk_b, semb)
            pltpu.make_async_copy(
                tab_hbm.at[:, :, pl.ds(pl.multiple_of(c0, 128), CHUNK)],
                chunk_a, sema).wait()
            do_extract(chunk_a, c0, CHUNK, None)
            cpa2 = start(c0 + 2 * CHUNK, chunk_a, sema)
            del cpa2
            cpb.wait()
            do_extract(chunk_b, c0 + CHUNK, CHUNK, None)
            return carry

        lax.fori_loop(0, CPW // 2, pair_iter, 0)
        pltpu.make_async_copy(
            tab_hbm.at[:, :, pl.ds(pl.multiple_of(
                lo + (CPW - 1) * CHUNK, 128), CHUNK)],
            chunk_a, sema).wait()
        do_extract(chunk_a, lo + (CPW - 1) * CHUNK, CHUNK, None)

        @pl.when(is_last)
        def _():
            cp = start(jnp.int32(FULL - CHUNK), chunk_a, sema)
            do_extract(chunk_a, jnp.int32(FULL - CHUNK), CHUNK, cp)
            do_extract(tail_v, jnp.int32(TAILLO), NROW - TAILLO, None)

    one_table(uidx_hbm, ut3_hbm, utail_v, uemb_hbm)
    one_table(iidx_hbm, it3_hbm, itail_v, iemb_hbm)


@functools.cache
def _build_compute():
    return pl.kernel(
        _compute_body,
        mesh=_mesh(),
        compiler_params=pltpu.CompilerParams(
            needs_layout_passes=False, use_tc_tiling_on_sc=True),
        out_type=jax.ShapeDtypeStruct((BATCH,), jnp.float32),
        scratch_types=[
            pltpu.VMEM((128, 128), jnp.float32),
            pltpu.VMEM((128, 128), jnp.float32),
            pltpu.VMEM((128,), jnp.float32),
            pltpu.VMEM((BPW,), jnp.float32),
            pltpu.SemaphoreType.DMA,
        ],
    )


def _compute_body(uemb_hbm, iemb_hbm, wb_hbm, out_hbm,
                  ub_v, ib_v, wb_v, out_v, sem):
    wid = lax.axis_index("s") * NC + lax.axis_index("c")
    base = wid * BPW
    pltpu.sync_copy(wb_hbm, wb_v)

    lane = lax.iota(jnp.int32, 16)
    w_lo = wb_v[pl.ds(0, 16)]
    w_hi = wb_v[pl.ds(16, 16)]
    bias = wb_v[pl.ds(DIM, 16)][0]

    def blk_body(blk, carry):
        b0 = base + blk * 128
        b0a = pl.multiple_of(b0, 128)
        cu = pltpu.async_copy(uemb_hbm.at[pl.ds(b0a, 128)], ub_v, sem)
        ci = pltpu.async_copy(iemb_hbm.at[pl.ds(b0a, 128)], ib_v, sem)
        cu.wait()
        ci.wait()

        def grp(g, carry2):
            acc = jnp.zeros((16,), jnp.float32)
            for k in range(16):
                r = g * 16 + k
                t = (ub_v[r, pl.ds(0, 16)] * ib_v[r, pl.ds(0, 16)]) * w_lo \
                    + (ub_v[r, pl.ds(16, 16)] * ib_v[r, pl.ds(16, 16)]) * w_hi
                acc = jnp.where(lane == k, jnp.sum(t), acc)
            out_v[pl.ds(blk * 128 + g * 16, 16)] = acc + bias
            return carry2

        lax.fori_loop(0, 8, grp, 0)
        return carry

    lax.fori_loop(0, BPW // 128, blk_body, 0)

    pltpu.sync_copy(out_v, out_hbm.at[pl.ds(base, BPW)])


def kernel(user_idx, item_idx, user_table, item_table, W, b):
    uidx = user_idx.astype(jnp.int32)
    iidx = item_idx.astype(jnp.int32)
    ut3 = user_table.T.reshape(4, 8, NROW)
    it3 = item_table.T.reshape(4, 8, NROW)
    utail = jnp.pad(user_table[FULL:, :].T, ((0, 0), (0, 128 - (NROW - FULL)))
                    ).reshape(4, 8, 128)
    itail = jnp.pad(item_table[FULL:, :].T, ((0, 0), (0, 128 - (NROW - FULL)))
                    ).reshape(4, 8, 128)
    wb = jnp.concatenate(
        [W.reshape(-1).astype(jnp.float32),
         b.reshape(-1).astype(jnp.float32),
         jnp.zeros((128 - DIM - 1,), jnp.float32)])
    uemb, iemb = _build_sweep()(uidx, iidx, ut3, it3, utail, itail)
    out = _build_compute()(uemb, iemb, wb)
    return out.reshape(BATCH, 1)


# speedup vs baseline: 1.1405x; 1.1405x over previous
import functools

import jax
import jax.numpy as jnp
from jax import lax
from jax.experimental import pallas as pl
from jax.experimental.pallas import tpu as pltpu
from jax.experimental.pallas import tpu_sc as plsc

DIM = 32
BATCH = 16384
NROW = 1000000

NC = 2
NS = 16
NW = NC * NS
BPW = BATCH // NW

CHUNK = 512
CPW = 61
FULL = 999936
TAILLO = FULL
PAD0 = BATCH
NSTG = BATCH + NW * 16


def _mesh():
    return plsc.VectorSubcoreMesh(
        core_axis_name="c", subcore_axis_name="s",
        num_cores=NC, num_subcores=NS)


@functools.cache
def _build_sweep():
    return pl.kernel(
        _sweep_body,
        mesh=_mesh(),
        compiler_params=pltpu.CompilerParams(
            needs_layout_passes=False, use_tc_tiling_on_sc=True),
        out_type=(jax.ShapeDtypeStruct((NSTG, 128), jnp.float32),
                  jax.ShapeDtypeStruct((NSTG, 128), jnp.float32)),
        scratch_types=[
            pltpu.VMEM((BATCH,), jnp.int32),
            pltpu.VMEM((BATCH + 16,), jnp.int32),
            pltpu.VMEM((BATCH + 16,), jnp.int32),
            pltpu.VMEM((BATCH + 16,), jnp.int32),
            pltpu.VMEM((BATCH + 16,), jnp.int32),
            pltpu.VMEM((4, 8, CHUNK), jnp.float32),
            pltpu.VMEM((4, 8, CHUNK), jnp.float32),
            pltpu.VMEM((4, 8, 128), jnp.float32),
            pltpu.VMEM((4, 8, 128), jnp.float32),
            pltpu.VMEM((16,), jnp.int32),
            pltpu.VMEM((512,), jnp.float32),
            pltpu.VMEM((16, 128), jnp.float32),
            pltpu.SemaphoreType.DMA,
            pltpu.SemaphoreType.DMA,
            pltpu.SemaphoreType.DMA,
        ],
    )


def _sweep_body(uidx_hbm, iidx_hbm, ut3_hbm, it3_hbm, utail_hbm, itail_hbm,
                uemb_hbm, iemb_hbm,
                idx_v, hitl_v, hitp_v, chl_v, chp_v, chunk_a, chunk_b,
                utail_v, itail_v, posb_v, trans_v, rows_v, sem, sema, semb):
    wid = lax.axis_index("s") * NC + lax.axis_index("c")
    lane16 = lax.iota(jnp.int32, 16)
    pad_pos = PAD0 + wid * 16 + lane16
    lo = wid * (CPW * CHUNK)
    is_last = wid == NW - 1
    hi = jnp.where(is_last, NROW, lo + CPW * CHUNK)

    pltpu.sync_copy(utail_hbm, utail_v)
    pltpu.sync_copy(itail_hbm, itail_v)

    def one_table(idx_hbm, tab_hbm, tail_v, emb_hbm):
        pltpu.sync_copy(idx_hbm, idx_v)

        def filt(t, cnt):
            vec = idx_v[pl.ds(t * 16, 16)]
            m = (vec >= lo) & (vec < hi)
            plsc.store_compressed(hitl_v.at[pl.ds(cnt, 16)], vec, mask=m)
            plsc.store_compressed(
                hitp_v.at[pl.ds(cnt, 16)], t * 16 + lane16, mask=m)
            return cnt + plsc.all_reduce_population_count(m)[0]

        cnt = lax.fori_loop(0, BATCH // 16, filt, jnp.int32(0))
        hitl_v[pl.ds(cnt, 16)] = jnp.broadcast_to(jnp.int32(lo), (16,))
        hitp_v[pl.ds(cnt, 16)] = pad_pos

        def do_extract(src_v, c0, width, cp):
            ngrp = (cnt + 15) // 16

            def refil(g, ccnt):
                v = hitl_v[pl.ds(g * 16, 16)]
                p = hitp_v[pl.ds(g * 16, 16)]
                m = (v >= c0) & (v < c0 + width)
                plsc.store_compressed(
                    chl_v.at[pl.ds(ccnt, 16)], v - c0, mask=m)
                plsc.store_compressed(chp_v.at[pl.ds(ccnt, 16)], p, mask=m)
                return ccnt + plsc.all_reduce_population_count(m)[0]

            ccnt = lax.fori_loop(0, ngrp, refil, jnp.int32(0))
            chl_v[pl.ds(ccnt, 16)] = jnp.zeros((16,), jnp.int32)
            chp_v[pl.ds(ccnt, 16)] = pad_pos
            if cp is not None:
                cp.wait()

            d16 = lane16 * 16

            def ext(h, carry):
                lvec = chl_v[pl.ds(h * 16, 16)]
                posb_v[pl.ds(0, 16)] = chp_v[pl.ds(h * 16, 16)]
                for d in range(DIM):
                    i, s = divmod(d, 8)
                    ivec = jnp.full((16,), i, jnp.int32)
                    svec = jnp.full((16,), s, jnp.int32)
                    vals = plsc.load_gather(src_v, [ivec, svec, lvec])
                    trans_v[pl.ds(d * 16, 16)] = vals
                for k in range(16):
                    rows_v[k, pl.ds(0, 16)] = plsc.load_gather(
                        trans_v, [d16 + k])
                    rows_v[k, pl.ds(16, 16)] = plsc.load_gather(
                        trans_v, [d16 + (256 + k)])
                pltpu.async_copy(rows_v, emb_hbm.at[posb_v], sem).wait()
                return carry

            lax.fori_loop(0, (ccnt + 15) // 16, ext, 0)

        def start(c0, buf, s):
            c0a = pl.multiple_of(c0, 128)
            return pltpu.async_copy(
                tab_hbm.at[:, :, pl.ds(c0a, CHUNK)], buf, s)

        start(lo, chunk_a, sema)

        def waiter(c0, buf, s):
            return pltpu.make_async_copy(
                tab_hbm.at[:, :, pl.ds(pl.multiple_of(c0, 128), CHUNK)],
                buf, s)

        def pair_iter(g, carry):
            c0 = lo + (2 * g) * CHUNK
            cpb = start(c0 + CHUNK, chunk_b, semb)
            do_extract(chunk_a, c0, CHUNK, waiter(c0, chunk_a, sema))
            cpa2 = start(c0 + 2 * CHUNK, chunk_a, sema)
            del cpa2
            do_extract(chunk_b, c0 + CHUNK, CHUNK, cpb)
            return carry

        lax.fori_loop(0, CPW // 2, pair_iter, 0)
        do_extract(chunk_a, lo + (CPW - 1) * CHUNK, CHUNK,
                   waiter(lo + (CPW - 1) * CHUNK, chunk_a, sema))

        @pl.when(is_last)
        def _():
            cp = start(jnp.int32(FULL - CHUNK), chunk_a, sema)
            do_extract(chunk_a, jnp.int32(FULL - CHUNK), CHUNK, cp)
            do_extract(tail_v, jnp.int32(TAILLO), NROW - TAILLO, None)

    one_table(uidx_hbm, ut3_hbm, utail_v, uemb_hbm)
    one_table(iidx_hbm, it3_hbm, itail_v, iemb_hbm)


@functools.cache
def _build_compute():
    return pl.kernel(
        _compute_body,
        mesh=_mesh(),
        compiler_params=pltpu.CompilerParams(
            needs_layout_passes=False, use_tc_tiling_on_sc=True),
        out_type=jax.ShapeDtypeStruct((BATCH,), jnp.float32),
        scratch_types=[
            pltpu.VMEM((128, 128), jnp.float32),
            pltpu.VMEM((128, 128), jnp.float32),
            pltpu.VMEM((128,), jnp.float32),
            pltpu.VMEM((BPW,), jnp.float32),
            pltpu.SemaphoreType.DMA,
        ],
    )


def _compute_body(uemb_hbm, iemb_hbm, wb_hbm, out_hbm,
                  ub_v, ib_v, wb_v, out_v, sem):
    wid = lax.axis_index("s") * NC + lax.axis_index("c")
    base = wid * BPW
    pltpu.sync_copy(wb_hbm, wb_v)

    lane = lax.iota(jnp.int32, 16)
    w_lo = wb_v[pl.ds(0, 16)]
    w_hi = wb_v[pl.ds(16, 16)]
    bias = wb_v[pl.ds(DIM, 16)][0]

    def blk_body(blk, carry):
        b0 = base + blk * 128
        b0a = pl.multiple_of(b0, 128)
        cu = pltpu.async_copy(uemb_hbm.at[pl.ds(b0a, 128)], ub_v, sem)
        ci = pltpu.async_copy(iemb_hbm.at[pl.ds(b0a, 128)], ib_v, sem)
        cu.wait()
        ci.wait()

        def grp(g, carry2):
            acc = jnp.zeros((16,), jnp.float32)
            for k in range(16):
                r = g * 16 + k
                t = (ub_v[r, pl.ds(0, 16)] * ib_v[r, pl.ds(0, 16)]) * w_lo \
                    + (ub_v[r, pl.ds(16, 16)] * ib_v[r, pl.ds(16, 16)]) * w_hi
                acc = jnp.where(lane == k, jnp.sum(t), acc)
            out_v[pl.ds(blk * 128 + g * 16, 16)] = acc + bias
            return carry2

        lax.fori_loop(0, 8, grp, 0)
        return carry

    lax.fori_loop(0, BPW // 128, blk_body, 0)

    pltpu.sync_copy(out_v, out_hbm.at[pl.ds(base, BPW)])


def kernel(user_idx, item_idx, user_table, item_table, W, b):
    uidx = user_idx.astype(jnp.int32)
    iidx = item_idx.astype(jnp.int32)
    ut3 = user_table.T.reshape(4, 8, NROW)
    it3 = item_table.T.reshape(4, 8, NROW)
    utail = jnp.pad(user_table[FULL:, :].T, ((0, 0), (0, 128 - (NROW - FULL)))
                    ).reshape(4, 8, 128)
    itail = jnp.pad(item_table[FULL:, :].T, ((0, 0), (0, 128 - (NROW - FULL)))
                    ).reshape(4, 8, 128)
    wb = jnp.concatenate(
        [W.reshape(-1).astype(jnp.float32),
         b.reshape(-1).astype(jnp.float32),
         jnp.zeros((128 - DIM - 1,), jnp.float32)])
    uemb, iemb = _build_sweep()(uidx, iidx, ut3, it3, utail, itail)
    out = _build_compute()(uemb, iemb, wb)
    return out.reshape(BATCH, 1)
